# baseline (speedup 1.0000x reference)
import functools

import jax
import jax.numpy as jnp
from jax import lax
from jax.experimental import pallas as pl
from jax.experimental.pallas import tpu as pltpu
from jax.experimental.pallas import tpu_sc as plsc

N_CTX = 16
CTX_DIM = 512
SEQ_LEN = 77
BATCH = 1024
N_SUF = SEQ_LEN - 1 - N_CTX

_info = plsc.get_sparse_core_info()
NC = _info.num_cores
NS = _info.num_subcores
NW = NC * NS
BPW = BATCH // NW


def _sc_body(cls3, cls1d, ctx_hbm, pre_hbm, suf_hbm, tok_hbm,
             out_hbm, otok_hbm,
             idxs_v, idxflat_v, row_v, tokbuf_v, sem, tsem):
    wid = lax.axis_index("s") * NC + lax.axis_index("c")
    base = wid * BPW
    pltpu.sync_copy(cls3.at[wid], idxs_v)
    pltpu.sync_copy(cls1d.at[pl.ds(base, BPW)], idxflat_v)
    ct = pltpu.async_copy(tok_hbm.at[idxflat_v], tokbuf_v, tsem)
    for k in range(BPW):
        idx1 = idxs_v.at[k]
        c1 = pltpu.async_copy(pre_hbm.at[idx1], row_v.at[:, pl.ds(0, 1)], sem)
        c2 = pltpu.async_copy(ctx_hbm.at[idx1], row_v.at[:, pl.ds(1, N_CTX)], sem)
        c3 = pltpu.async_copy(suf_hbm.at[idx1], row_v.at[:, pl.ds(1 + N_CTX, N_SUF)], sem)
        c1.wait()
        c2.wait()
        c3.wait()
        pltpu.sync_copy(row_v, out_hbm.at[pl.ds(base + k, 1)])
    ct.wait()
    pltpu.sync_copy(tokbuf_v, otok_hbm.at[pl.ds(base, BPW)])


TOK_PAD = 80


@jax.jit
def _sc_gather(cls3, cls1d, ctx_pos, token_prefix_pos, token_suffix_pos, tokenized_prompts):
    f = functools.partial(
        pl.kernel,
        mesh=plsc.VectorSubcoreMesh(core_axis_name="c", subcore_axis_name="s"),
        out_type=(
            jax.ShapeDtypeStruct((BATCH, SEQ_LEN, CTX_DIM), jnp.float32),
            jax.ShapeDtypeStruct((BATCH, TOK_PAD), jnp.int32),
        ),
        scratch_types=[
            pltpu.VMEM((BPW, 1), jnp.int32),
            pltpu.VMEM((BPW,), jnp.int32),
            pltpu.VMEM((1, SEQ_LEN, CTX_DIM), jnp.float32),
            pltpu.VMEM((BPW, TOK_PAD), jnp.int32),
            pltpu.SemaphoreType.DMA,
            pltpu.SemaphoreType.DMA,
        ],
        compiler_params=pltpu.CompilerParams(use_tc_tiling_on_sc=False),
    )(_sc_body)
    return f(cls3, cls1d, ctx_pos, token_prefix_pos, token_suffix_pos, tokenized_prompts)


def kernel(cls_id, ctx_pos, token_prefix_pos, token_suffix_pos, tokenized_prompts):
    cls3 = cls_id.reshape(NW, BPW, 1)
    tok_pad = jnp.pad(tokenized_prompts, ((0, 0), (0, TOK_PAD - SEQ_LEN)))
    prompts, tok = _sc_gather(cls3, cls_id, ctx_pos, token_prefix_pos,
                              token_suffix_pos, tok_pad)
    return prompts, tok[:, :SEQ_LEN]


# speedup vs baseline: 3.9698x; 3.9698x over previous
import functools

import jax
import jax.numpy as jnp
from jax import lax
from jax.experimental import pallas as pl
from jax.experimental.pallas import tpu as pltpu
from jax.experimental.pallas import tpu_sc as plsc

N_CTX = 16
CTX_DIM = 512
SEQ_LEN = 77
BATCH = 1024
N_SUF = SEQ_LEN - 1 - N_CTX
TOK_PAD = 128

_info = plsc.get_sparse_core_info()
NC = _info.num_cores
NS = _info.num_subcores
NW = NC * NS
BPW = BATCH // NW
NBUF = 2


def _sc_body(cls1d, ctx_hbm, pre_hbm, suf_hbm, tok_hbm,
             gctx_hbm, gpre_hbm, gsuf_hbm, gtok_hbm,
             idxflat_v, ctx_v, pre_v, suf_v, tokbuf_v,
             gsem0, gsem1, wsem0, wsem1, tsem):
    wid = lax.axis_index("s") * NC + lax.axis_index("c")
    base = wid * BPW
    gsems = [gsem0, gsem1]
    wsems = [wsem0, wsem1]
    pltpu.sync_copy(cls1d.at[pl.ds(base, BPW)], idxflat_v)
    ct = pltpu.async_copy(tok_hbm.at[idxflat_v], tokbuf_v, tsem)

    ivs = [idxflat_v[pl.ds(16 * g, 16)] for g in range(BPW // 16)]

    gh = [None] * NBUF
    wh = [None] * NBUF

    def flush(k):
        j = k % NBUF
        for h in gh[j]:
            h.wait()
        gb = base + k
        wh[j] = (
            pltpu.async_copy(pre_v.at[j], gpre_hbm.at[pl.ds(gb, 1)], wsems[j]),
            pltpu.async_copy(ctx_v.at[j], gctx_hbm.at[pl.ds(gb, 1)], wsems[j]),
            pltpu.async_copy(suf_v.at[j], gsuf_hbm.at[pl.ds(gb, 1)], wsems[j]),
        )

    for k in range(BPW):
        j = k % NBUF
        if wh[j] is not None:
            for h in wh[j]:
                h.wait()
        c = ivs[k // 16][k % 16]
        gh[j] = (
            pltpu.async_copy(pre_hbm.at[pl.ds(c, 1)], pre_v.at[j], gsems[j]),
            pltpu.async_copy(ctx_hbm.at[pl.ds(c, 1)], ctx_v.at[j], gsems[j]),
            pltpu.async_copy(suf_hbm.at[pl.ds(c, 1)], suf_v.at[j], gsems[j]),
        )
        if k >= 1:
            flush(k - 1)
    flush(BPW - 1)
    for j in range(NBUF):
        if wh[j] is not None:
            for h in wh[j]:
                h.wait()
    ct.wait()
    pltpu.sync_copy(tokbuf_v, gtok_hbm.at[pl.ds(base, BPW)])


def _sc_gather(cls1d, ctx_pos, token_prefix_pos, token_suffix_pos, tok_pad):
    f = functools.partial(
        pl.kernel,
        mesh=plsc.VectorSubcoreMesh(core_axis_name="c", subcore_axis_name="s"),
        out_type=(
            jax.ShapeDtypeStruct((BATCH, N_CTX, CTX_DIM), jnp.float32),
            jax.ShapeDtypeStruct((BATCH, 1, CTX_DIM), jnp.float32),
            jax.ShapeDtypeStruct((BATCH, N_SUF, CTX_DIM), jnp.float32),
            jax.ShapeDtypeStruct((BATCH, TOK_PAD), jnp.int32),
        ),
        scratch_types=[
            pltpu.VMEM((BPW,), jnp.int32),
            pltpu.VMEM((NBUF, 1, N_CTX, CTX_DIM), jnp.float32),
            pltpu.VMEM((NBUF, 1, 1, CTX_DIM), jnp.float32),
            pltpu.VMEM((NBUF, 1, N_SUF, CTX_DIM), jnp.float32),
            pltpu.VMEM((BPW, TOK_PAD), jnp.int32),
            pltpu.SemaphoreType.DMA,
            pltpu.SemaphoreType.DMA,
            pltpu.SemaphoreType.DMA,
            pltpu.SemaphoreType.DMA,
            pltpu.SemaphoreType.DMA,
        ],
    )(_sc_body)
    return f(cls1d, ctx_pos, token_prefix_pos, token_suffix_pos, tok_pad)


def _tc_concat_body(pre_ref, ctx_ref, suf_ref, out_ref):
    out_ref[:, 0:1, :] = pre_ref[...]
    out_ref[:, 1:1 + N_CTX, :] = ctx_ref[...]
    out_ref[:, 1 + N_CTX:SEQ_LEN, :] = suf_ref[...]


_TC_BS = 8


def _tc_concat(g_pre, g_ctx, g_suf):
    return pl.pallas_call(
        _tc_concat_body,
        grid=(BATCH // _TC_BS,),
        in_specs=[
            pl.BlockSpec((_TC_BS, 1, CTX_DIM), lambda i: (i, 0, 0)),
            pl.BlockSpec((_TC_BS, N_CTX, CTX_DIM), lambda i: (i, 0, 0)),
            pl.BlockSpec((_TC_BS, N_SUF, CTX_DIM), lambda i: (i, 0, 0)),
        ],
        out_specs=pl.BlockSpec((_TC_BS, SEQ_LEN, CTX_DIM), lambda i: (i, 0, 0)),
        out_shape=jax.ShapeDtypeStruct((BATCH, SEQ_LEN, CTX_DIM), jnp.float32),
    )(g_pre, g_ctx, g_suf)


@jax.jit
def _run(cls_id, ctx_pos, token_prefix_pos, token_suffix_pos, tokenized_prompts):
    tok_pad = jnp.pad(tokenized_prompts, ((0, 0), (0, TOK_PAD - SEQ_LEN)))
    g_ctx, g_pre, g_suf, g_tok = _sc_gather(
        cls_id, ctx_pos, token_prefix_pos, token_suffix_pos, tok_pad)
    prompts = _tc_concat(g_pre, g_ctx, g_suf)
    return prompts, g_tok[:, :SEQ_LEN]


def kernel(cls_id, ctx_pos, token_prefix_pos, token_suffix_pos, tokenized_prompts):
    return _run(cls_id, ctx_pos, token_prefix_pos, token_suffix_pos,
                tokenized_prompts)


# speedup vs baseline: 4.0810x; 1.0280x over previous
import functools

import jax
import jax.numpy as jnp
from jax import lax
from jax.experimental import pallas as pl
from jax.experimental.pallas import tpu as pltpu
from jax.experimental.pallas import tpu_sc as plsc

N_CTX = 16
CTX_DIM = 512
SEQ_LEN = 77
BATCH = 1024
N_SUF = SEQ_LEN - 1 - N_CTX
TOK_PAD = 128

_info = plsc.get_sparse_core_info()
NC = _info.num_cores
NS = _info.num_subcores
NW = NC * NS
BPW = BATCH // NW
NBUF = 2


def _sc_body(cls1d, ctx_hbm, pre_hbm, suf_hbm, tok_hbm,
             gctx_hbm, gpre_hbm, gsuf_hbm, gtok_hbm,
             idxflat_v, ctx_v, pre_v, suf_v, tokbuf_v,
             gsem0, gsem1, wsem0, wsem1, tsem):
    wid = lax.axis_index("s") * NC + lax.axis_index("c")
    base = wid * BPW
    gsems = [gsem0, gsem1]
    wsems = [wsem0, wsem1]
    pltpu.sync_copy(cls1d.at[pl.ds(base, BPW)], idxflat_v)
    ct = pltpu.async_copy(tok_hbm.at[idxflat_v], tokbuf_v, tsem)

    ivs = [idxflat_v[pl.ds(16 * g, 16)] for g in range(BPW // 16)]

    gh = [None] * NBUF
    wh = [None] * NBUF

    def flush(k):
        j = k % NBUF
        for h in gh[j]:
            h.wait()
        gb = base + k
        wh[j] = (
            pltpu.async_copy(pre_v.at[j], gpre_hbm.at[pl.ds(gb, 1)], wsems[j]),
            pltpu.async_copy(ctx_v.at[j], gctx_hbm.at[pl.ds(gb, 1)], wsems[j]),
            pltpu.async_copy(suf_v.at[j], gsuf_hbm.at[pl.ds(gb, 1)], wsems[j]),
        )

    for k in range(BPW):
        j = k % NBUF
        if wh[j] is not None:
            for h in wh[j]:
                h.wait()
        c = ivs[k // 16][k % 16]
        gh[j] = (
            pltpu.async_copy(pre_hbm.at[pl.ds(c, 1)], pre_v.at[j], gsems[j]),
            pltpu.async_copy(ctx_hbm.at[pl.ds(c, 1)], ctx_v.at[j], gsems[j]),
            pltpu.async_copy(suf_hbm.at[pl.ds(c, 1)], suf_v.at[j], gsems[j]),
        )
        if k >= 1:
            flush(k - 1)
    flush(BPW - 1)
    for j in range(NBUF):
        if wh[j] is not None:
            for h in wh[j]:
                h.wait()
    ct.wait()
    pltpu.sync_copy(tokbuf_v, gtok_hbm.at[pl.ds(base, BPW)])


def _sc_gather(cls1d, ctx_pos, token_prefix_pos, token_suffix_pos, tok_pad):
    f = functools.partial(
        pl.kernel,
        mesh=plsc.VectorSubcoreMesh(core_axis_name="c", subcore_axis_name="s"),
        out_type=(
            jax.ShapeDtypeStruct((BATCH, N_CTX, CTX_DIM), jnp.float32),
            jax.ShapeDtypeStruct((BATCH, 1, CTX_DIM), jnp.float32),
            jax.ShapeDtypeStruct((BATCH, N_SUF, CTX_DIM), jnp.float32),
            jax.ShapeDtypeStruct((BATCH, TOK_PAD), jnp.int32),
        ),
        scratch_types=[
            pltpu.VMEM((BPW,), jnp.int32),
            pltpu.VMEM((NBUF, 1, N_CTX, CTX_DIM), jnp.float32),
            pltpu.VMEM((NBUF, 1, 1, CTX_DIM), jnp.float32),
            pltpu.VMEM((NBUF, 1, N_SUF, CTX_DIM), jnp.float32),
            pltpu.VMEM((BPW, TOK_PAD), jnp.int32),
            pltpu.SemaphoreType.DMA,
            pltpu.SemaphoreType.DMA,
            pltpu.SemaphoreType.DMA,
            pltpu.SemaphoreType.DMA,
            pltpu.SemaphoreType.DMA,
        ],
    )(_sc_body)
    return f(cls1d, ctx_pos, token_prefix_pos, token_suffix_pos, tok_pad)


def _tc_concat_body(pre_ref, ctx_ref, suf_ref, out_ref):
    out_ref[...] = jnp.concatenate(
        [pre_ref[...], ctx_ref[...], suf_ref[...]], axis=1)


_TC_BS = 32


def _tc_concat(g_pre, g_ctx, g_suf):
    return pl.pallas_call(
        _tc_concat_body,
        grid=(BATCH // _TC_BS,),
        in_specs=[
            pl.BlockSpec((_TC_BS, 1, CTX_DIM), lambda i: (i, 0, 0)),
            pl.BlockSpec((_TC_BS, N_CTX, CTX_DIM), lambda i: (i, 0, 0)),
            pl.BlockSpec((_TC_BS, N_SUF, CTX_DIM), lambda i: (i, 0, 0)),
        ],
        out_specs=pl.BlockSpec((_TC_BS, SEQ_LEN, CTX_DIM), lambda i: (i, 0, 0)),
        out_shape=jax.ShapeDtypeStruct((BATCH, SEQ_LEN, CTX_DIM), jnp.float32),
    )(g_pre, g_ctx, g_suf)


@jax.jit
def _run(cls_id, ctx_pos, token_prefix_pos, token_suffix_pos, tokenized_prompts):
    tok_pad = jnp.pad(tokenized_prompts, ((0, 0), (0, TOK_PAD - SEQ_LEN)))
    g_ctx, g_pre, g_suf, g_tok = _sc_gather(
        cls_id, ctx_pos, token_prefix_pos, token_suffix_pos, tok_pad)
    prompts = _tc_concat(g_pre, g_ctx, g_suf)
    return prompts, g_tok[:, :SEQ_LEN]


def kernel(cls_id, ctx_pos, token_prefix_pos, token_suffix_pos, tokenized_prompts):
    return _run(cls_id, ctx_pos, token_prefix_pos, token_suffix_pos,
                tokenized_prompts)
